# Initial kernel scaffold; baseline (speedup 1.0000x reference)
import functools

import jax
import jax.numpy as jnp
from jax import lax
from jax.experimental import pallas as pl
from jax.experimental.pallas import tpu as pltpu
from jax.experimental.pallas import tpu_sc as plsc

N = 10000
E = 320000
F_IN = 128
F_OUT = 64
NPAD = 10240
RPT = NPAD // 16
K = 128
NCHUNK = E // K
NW = 32
ITERS = (NCHUNK + NW - 1) // NW

_mesh = plsc.VectorSubcoreMesh(core_axis_name="c", subcore_axis_name="s")


@functools.partial(
    pl.kernel,
    mesh=_mesh,
    out_type=(
        jax.ShapeDtypeStruct((NPAD,), jnp.float32),
        jax.ShapeDtypeStruct((NPAD,), jnp.float32),
    ),
    scratch_types=[
        pltpu.VMEM((K,), jnp.int32),
        pltpu.VMEM((K,), jnp.float32),
        pltpu.VMEM_SHARED((NPAD,), jnp.float32),
    ],
)
def _deg_kernel(dst_hbm, ones_hbm, zeros_hbm, d0_hbm, d1_hbm, didx, ones_v, deg_s):
    c = lax.axis_index("c")
    s = lax.axis_index("s")
    wid = c * 16 + s
    pltpu.sync_copy(ones_hbm, ones_v)
    pltpu.sync_copy(zeros_hbm, deg_s.at[pl.ds(s * RPT, RPT)])
    plsc.subcore_barrier()

    def body(i, carry):
        cid = wid + NW * i

        @pl.when(cid < NCHUNK)
        def _():
            pltpu.sync_copy(dst_hbm.at[pl.ds(cid * K, K)], didx)
            pltpu.sync_copy(ones_v, deg_s.at[didx], add=True)

        return carry

    lax.fori_loop(0, ITERS, body, 0)
    plsc.subcore_barrier()

    @pl.when(c == 0)
    def _():
        pltpu.sync_copy(deg_s.at[pl.ds(s * RPT, RPT)], d0_hbm.at[pl.ds(s * RPT, RPT)])

    @pl.when(c == 1)
    def _():
        pltpu.sync_copy(deg_s.at[pl.ds(s * RPT, RPT)], d1_hbm.at[pl.ds(s * RPT, RPT)])


@functools.partial(
    pl.kernel,
    mesh=_mesh,
    out_type=(
        jax.ShapeDtypeStruct((NPAD, F_OUT), jnp.float32),
        jax.ShapeDtypeStruct((NPAD, F_OUT), jnp.float32),
    ),
    scratch_types=[
        pltpu.VMEM((K,), jnp.int32),
        pltpu.VMEM((K,), jnp.int32),
        pltpu.VMEM((K, F_OUT), jnp.float32),
        pltpu.VMEM_SHARED((NPAD, F_OUT), jnp.float32),
        pltpu.SemaphoreType.DMA,
    ],
)
def _agg_kernel(hs_hbm, src_hbm, dst_hbm, z2_hbm, p0_hbm, p1_hbm,
                sidx, didx, rows, part, sem):
    c = lax.axis_index("c")
    s = lax.axis_index("s")
    wid = c * 16 + s
    pltpu.sync_copy(z2_hbm, part.at[pl.ds(s * RPT, RPT)])
    plsc.subcore_barrier()

    def body(i, carry):
        cid = wid + NW * i

        @pl.when(cid < NCHUNK)
        def _():
            pltpu.sync_copy(src_hbm.at[pl.ds(cid * K, K)], sidx)
            pltpu.sync_copy(dst_hbm.at[pl.ds(cid * K, K)], didx)
            pltpu.async_copy(hs_hbm.at[sidx], rows, sem).wait()
            pltpu.sync_copy(rows, part.at[didx], add=True)

        return carry

    lax.fori_loop(0, ITERS, body, 0)
    plsc.subcore_barrier()

    @pl.when(c == 0)
    def _():
        pltpu.sync_copy(part.at[pl.ds(s * RPT, RPT)], p0_hbm.at[pl.ds(s * RPT, RPT)])

    @pl.when(c == 1)
    def _():
        pltpu.sync_copy(part.at[pl.ds(s * RPT, RPT)], p1_hbm.at[pl.ds(s * RPT, RPT)])


BLK = 1000


def _mm_body(x_ref, w_ref, d0_ref, d1_ref, hs_ref, dinv_ref):
    deg = d0_ref[...] + d1_ref[...] + 1.0
    dinv = lax.rsqrt(deg)
    h = jnp.dot(x_ref[...], w_ref[...], preferred_element_type=jnp.float32)
    hs_ref[...] = h * dinv
    dinv_ref[...] = dinv


_mm = pl.pallas_call(
    _mm_body,
    grid=(N // BLK,),
    in_specs=[
        pl.BlockSpec((BLK, F_IN), lambda i: (i, 0)),
        pl.BlockSpec((F_IN, F_OUT), lambda i: (0, 0)),
        pl.BlockSpec((BLK, 1), lambda i: (i, 0)),
        pl.BlockSpec((BLK, 1), lambda i: (i, 0)),
    ],
    out_specs=[
        pl.BlockSpec((BLK, F_OUT), lambda i: (i, 0)),
        pl.BlockSpec((BLK, 1), lambda i: (i, 0)),
    ],
    out_shape=[
        jax.ShapeDtypeStruct((N, F_OUT), jnp.float32),
        jax.ShapeDtypeStruct((N, 1), jnp.float32),
    ],
)


def _final_body(p0_ref, p1_ref, hs_ref, dinv_ref, b_ref, out_ref):
    acc = p0_ref[...] + p1_ref[...] + hs_ref[...]
    out_ref[...] = jnp.maximum(acc * dinv_ref[...] + b_ref[...], 0.0)


_final = pl.pallas_call(
    _final_body,
    grid=(N // BLK,),
    in_specs=[
        pl.BlockSpec((BLK, F_OUT), lambda i: (i, 0)),
        pl.BlockSpec((BLK, F_OUT), lambda i: (i, 0)),
        pl.BlockSpec((BLK, F_OUT), lambda i: (i, 0)),
        pl.BlockSpec((BLK, 1), lambda i: (i, 0)),
        pl.BlockSpec((1, F_OUT), lambda i: (0, 0)),
    ],
    out_specs=pl.BlockSpec((BLK, F_OUT), lambda i: (i, 0)),
    out_shape=jax.ShapeDtypeStruct((N, F_OUT), jnp.float32),
)


def kernel(x, edge_index, W, b):
    src = edge_index[0]
    dst = edge_index[1]
    ones = jnp.ones((K,), jnp.float32)
    z1 = jnp.zeros((RPT,), jnp.float32)
    z2 = jnp.zeros((RPT, F_OUT), jnp.float32)
    d0, d1 = _deg_kernel(dst, ones, z1)
    hs, dinv = _mm(x, W, d0.reshape(NPAD, 1), d1.reshape(NPAD, 1))
    p0, p1 = _agg_kernel(hs, src, dst, z2)
    return _final(p0[:N], p1[:N], hs, dinv, b.reshape(1, F_OUT))


# speedup vs baseline: 24.2266x; 24.2266x over previous
import functools

import jax
import jax.numpy as jnp
from jax import lax
from jax.experimental import pallas as pl
from jax.experimental.pallas import tpu as pltpu
from jax.experimental.pallas import tpu_sc as plsc

N = 10000
E = 320000
F_IN = 128
F_OUT = 64
NPAD = 10240
RPT = NPAD // 16
K = 128
NCHUNK = E // K
NW = 32
ITERS = (NCHUNK + NW - 1) // NW

_mesh = plsc.VectorSubcoreMesh(core_axis_name="c", subcore_axis_name="s")


@functools.partial(
    pl.kernel,
    mesh=_mesh,
    out_type=(
        jax.ShapeDtypeStruct((NPAD,), jnp.float32),
        jax.ShapeDtypeStruct((NPAD,), jnp.float32),
    ),
    scratch_types=[
        pltpu.VMEM((K,), jnp.int32),
        pltpu.VMEM((K,), jnp.float32),
        pltpu.VMEM_SHARED((NPAD,), jnp.float32),
    ],
)
def _deg_kernel(dst_hbm, ones_hbm, zeros_hbm, d0_hbm, d1_hbm, didx, ones_v, deg_s):
    c = lax.axis_index("c")
    s = lax.axis_index("s")
    wid = c * 16 + s
    pltpu.sync_copy(ones_hbm, ones_v)
    pltpu.sync_copy(zeros_hbm, deg_s.at[pl.ds(s * RPT, RPT)])
    plsc.subcore_barrier()

    def body(i, carry):
        cid = wid + NW * i

        @pl.when(cid < NCHUNK)
        def _():
            pltpu.sync_copy(dst_hbm.at[pl.ds(cid * K, K)], didx)
            pltpu.sync_copy(ones_v, deg_s.at[didx], add=True)

        return carry

    lax.fori_loop(0, ITERS, body, 0)
    plsc.subcore_barrier()

    @pl.when(c == 0)
    def _():
        pltpu.sync_copy(deg_s.at[pl.ds(s * RPT, RPT)], d0_hbm.at[pl.ds(s * RPT, RPT)])

    @pl.when(c == 1)
    def _():
        pltpu.sync_copy(deg_s.at[pl.ds(s * RPT, RPT)], d1_hbm.at[pl.ds(s * RPT, RPT)])


@functools.partial(
    pl.kernel,
    mesh=_mesh,
    compiler_params=pltpu.CompilerParams(use_tc_tiling_on_sc=False),
    out_type=(
        jax.ShapeDtypeStruct((NPAD, F_OUT), jnp.float32),
        jax.ShapeDtypeStruct((NPAD, F_OUT), jnp.float32),
    ),
    scratch_types=[
        pltpu.VMEM((K,), jnp.int32),
        pltpu.VMEM((K,), jnp.int32),
        pltpu.VMEM((K, F_OUT), jnp.float32),
        pltpu.VMEM_SHARED((NPAD, F_OUT), jnp.float32),
        pltpu.SemaphoreType.DMA,
    ],
)
def _agg_kernel(hs_hbm, src_hbm, dst_hbm, z2_hbm, p0_hbm, p1_hbm,
                sidx, didx, rows, part, sem):
    c = lax.axis_index("c")
    s = lax.axis_index("s")
    wid = c * 16 + s
    pltpu.sync_copy(z2_hbm, part.at[pl.ds(s * RPT, RPT)])
    plsc.subcore_barrier()

    def body(i, carry):
        cid = wid + NW * i

        @pl.when(cid < NCHUNK)
        def _():
            pltpu.sync_copy(src_hbm.at[pl.ds(cid * K, K)], sidx)
            pltpu.sync_copy(dst_hbm.at[pl.ds(cid * K, K)], didx)
            pltpu.async_copy(hs_hbm.at[sidx], rows, sem).wait()
            pltpu.sync_copy(rows, part.at[didx], add=True)

        return carry

    lax.fori_loop(0, ITERS, body, 0)
    plsc.subcore_barrier()

    @pl.when(c == 0)
    def _():
        pltpu.sync_copy(part.at[pl.ds(s * RPT, RPT)], p0_hbm.at[pl.ds(s * RPT, RPT)])

    @pl.when(c == 1)
    def _():
        pltpu.sync_copy(part.at[pl.ds(s * RPT, RPT)], p1_hbm.at[pl.ds(s * RPT, RPT)])


BLK = 1000


def _mm_body(x_ref, w_ref, d0_ref, d1_ref, hs_ref, dinv_ref):
    deg = d0_ref[...] + d1_ref[...] + 1.0
    dinv = lax.rsqrt(deg)
    h = jnp.dot(x_ref[...], w_ref[...], preferred_element_type=jnp.float32)
    hs_ref[...] = h * dinv
    dinv_ref[...] = dinv


_mm = pl.pallas_call(
    _mm_body,
    grid=(N // BLK,),
    in_specs=[
        pl.BlockSpec((BLK, F_IN), lambda i: (i, 0)),
        pl.BlockSpec((F_IN, F_OUT), lambda i: (0, 0)),
        pl.BlockSpec((BLK, 1), lambda i: (i, 0)),
        pl.BlockSpec((BLK, 1), lambda i: (i, 0)),
    ],
    out_specs=[
        pl.BlockSpec((BLK, F_OUT), lambda i: (i, 0)),
        pl.BlockSpec((BLK, 1), lambda i: (i, 0)),
    ],
    out_shape=[
        jax.ShapeDtypeStruct((N, F_OUT), jnp.float32),
        jax.ShapeDtypeStruct((N, 1), jnp.float32),
    ],
)


def _final_body(p0_ref, p1_ref, hs_ref, dinv_ref, b_ref, out_ref):
    acc = p0_ref[...] + p1_ref[...] + hs_ref[...]
    out_ref[...] = jnp.maximum(acc * dinv_ref[...] + b_ref[...], 0.0)


_final = pl.pallas_call(
    _final_body,
    grid=(N // BLK,),
    in_specs=[
        pl.BlockSpec((BLK, F_OUT), lambda i: (i, 0)),
        pl.BlockSpec((BLK, F_OUT), lambda i: (i, 0)),
        pl.BlockSpec((BLK, F_OUT), lambda i: (i, 0)),
        pl.BlockSpec((BLK, 1), lambda i: (i, 0)),
        pl.BlockSpec((1, F_OUT), lambda i: (0, 0)),
    ],
    out_specs=pl.BlockSpec((BLK, F_OUT), lambda i: (i, 0)),
    out_shape=jax.ShapeDtypeStruct((N, F_OUT), jnp.float32),
)


def kernel(x, edge_index, W, b):
    src = edge_index[0]
    dst = edge_index[1]
    ones = jnp.ones((K,), jnp.float32)
    z1 = jnp.zeros((RPT,), jnp.float32)
    z2 = jnp.zeros((RPT, F_OUT), jnp.float32)
    d0, d1 = _deg_kernel(dst, ones, z1)
    hs, dinv = _mm(x, W, d0.reshape(NPAD, 1), d1.reshape(NPAD, 1))
    p0, p1 = _agg_kernel(hs, src, dst, z2)
    return _final(p0[:N], p1[:N], hs, dinv, b.reshape(1, F_OUT))
